# baseline (speedup 1.0000x reference)
import jax
import jax.numpy as jnp
from jax import lax
from jax.experimental import pallas as pl
from jax.experimental.pallas import tpu as pltpu
from jax.experimental.pallas import tpu_sc as plsc

_V = 1000000
_D = 64
_B = 4096
_L = 200

_NC = 2
_NS = 16
_NW = _NC * _NS
_LANES = 16
_NBLK = _V // 128
_TAIL = _V - _NBLK * 128
_BLK_PER_W = (_NBLK + 1 + _NW - 1) // _NW
_SEQ_W = _B // _NW

_MESH = dict(core_axis_name="c", subcore_axis_name="s",
             num_cores=_NC, num_subcores=_NS)
_PARAMS = pltpu.CompilerParams(
    use_tc_tiling_on_sc=True, needs_layout_passes=False)


def _wid():
  return lax.axis_index("s") * _NC + lax.axis_index("c")


def _iota16():
  return lax.iota(jnp.int32, 16)


def _splat(v):
  return jnp.zeros((16,), jnp.int32) + v


def _transpose_tokens(vin, vout, n_tok):
  base = _iota16()

  @pl.loop(0, _D)
  def _d(d):
    dcol = _splat(d)
    for q in range(n_tok // _LANES):
      v = vin[d, pl.ds(q * _LANES, _LANES)]
      plsc.store_scatter(vout, [base + q * _LANES, dcol], v)


def _t_body(tblT, tailP, tblL, vin, vout, sem):
  w = _wid()

  @pl.when(w == 0)
  def _tail():
    pltpu.sync_copy(tailP, tblL.at[pl.ds(_NBLK * 128, _TAIL)])

  @pl.loop(0, _BLK_PER_W)
  def _blk(k):
    b = w + k * _NW

    @pl.when(b < _NBLK)
    def _full():
      pltpu.async_copy(tblT.at[:, pl.ds(b * 128, 128)], vin, sem).wait()
      _transpose_tokens(vin, vout, 128)
      pltpu.sync_copy(vout, tblL.at[pl.ds(b * 128, 128)])


def _g_body(xT, tblL, posT, out, idx_v, posw, posv2, rows, slab, gsem):
  w = _wid()
  lane0 = w * _SEQ_W

  pltpu.sync_copy(xT.at[:, pl.ds(lane0, _SEQ_W)], idx_v)
  pltpu.sync_copy(posT, posw)
  base = _iota16()

  @pl.loop(0, _D)
  def _pd(d):
    dcol = _splat(d)
    for q in range(13):
      o = min(q * _LANES, _L - _LANES)
      v = posw[d, pl.ds(o, _LANES)]
      plsc.store_scatter(posv2, [base + o, dcol], v)

  @pl.loop(0, _L)
  def _pos(l):
    pltpu.async_copy(tblL.at[idx_v.at[l]], rows, gsem).wait()

    @pl.loop(0, _SEQ_W)
    def _tok(j):
      jcol = _splat(j)
      for q in range(_D // _LANES):
        v = rows[j, pl.ds(q * _LANES, _LANES)]
        p = posv2[l, pl.ds(q * _LANES, _LANES)]
        plsc.store_scatter(slab, [base + q * _LANES, jcol], v + p)

    pltpu.sync_copy(slab, out.at[l, :, pl.ds(lane0, _SEQ_W)])


@jax.jit
def _run(x, token_table, pos_table):
  mesh = plsc.VectorSubcoreMesh(**_MESH)
  tblT = token_table.T
  xT = x.T
  posT = pos_table.T
  tailP = jnp.pad(token_table[_NBLK * 128:], ((0, 0), (0, 128 - _D)))

  t_kern = pl.kernel(
      _t_body,
      out_type=jax.ShapeDtypeStruct((_V, 128), jnp.float32),
      mesh=mesh,
      scratch_types=[
          pltpu.VMEM((_D, 128), jnp.float32),
          pltpu.VMEM((128, 128), jnp.float32),
          pltpu.SemaphoreType.DMA,
      ],
      compiler_params=_PARAMS,
  )
  tblL = t_kern(tblT, tailP)

  g_kern = pl.kernel(
      _g_body,
      out_type=jax.ShapeDtypeStruct((_L, _D, _B), jnp.float32),
      mesh=mesh,
      scratch_types=[
          pltpu.VMEM((_L, _SEQ_W), jnp.int32),
          pltpu.VMEM((_D, _L), jnp.float32),
          pltpu.VMEM((_L, _D), jnp.float32),
          pltpu.VMEM((_SEQ_W, 128), jnp.float32),
          pltpu.VMEM((_D, _SEQ_W), jnp.float32),
          pltpu.SemaphoreType.DMA,
      ],
      compiler_params=_PARAMS,
  )
  outK = g_kern(xT, tblL, posT)
  return jnp.transpose(outK, (2, 0, 1))


def kernel(x, token_table, pos_table):
  return _run(x, token_table, pos_table[:_L])


# speedup vs baseline: 1.3574x; 1.3574x over previous
import jax
import jax.numpy as jnp
from jax import lax
from jax.experimental import pallas as pl
from jax.experimental.pallas import tpu as pltpu
from jax.experimental.pallas import tpu_sc as plsc

_V = 1000000
_D = 64
_B = 4096
_L = 200

_NC = 2
_NS = 16
_NW = _NC * _NS
_LANES = 16
_NBLK = _V // 128
_TAIL = _V - _NBLK * 128
_SLOT_PAIRS = (_NBLK // _NW + 2) // 2
_SEQ_W = _B // _NW

_MESH = dict(core_axis_name="c", subcore_axis_name="s",
             num_cores=_NC, num_subcores=_NS)
_PARAMS = pltpu.CompilerParams(
    use_tc_tiling_on_sc=True, needs_layout_passes=False)


def _wid():
  return lax.axis_index("s") * _NC + lax.axis_index("c")


def _row_bases():
  base = lax.iota(jnp.int32, _LANES)
  return [base + q * _LANES for q in range(8)]


def _t_body(tblT, tailP, tblL, vin, vout, gsem0, gsem1, wsem0, wsem1):
  w = _wid()
  gsems = (gsem0, gsem1)
  wsems = (wsem0, wsem1)
  rows_q = _row_bases()
  zeros = jnp.zeros((_LANES,), jnp.int32)

  @pl.when(w == 0)
  def _tail():
    pltpu.sync_copy(tailP, tblL.at[pl.ds(_NBLK * 128, _TAIL)])

  def issue(k, p):
    b = w + k * _NW

    @pl.when(b < _NBLK)
    def _():
      pltpu.async_copy(tblT.at[:, pl.ds(b * 128, 128)], vin.at[p], gsems[p])

  def process(k, p):
    b = w + k * _NW

    @pl.when(b < _NBLK)
    def _():
      pltpu.make_async_copy(
          tblT.at[:, pl.ds(0, 128)], vin.at[p], gsems[p]).wait()

      @pl.when(k >= 2)
      def _():
        pltpu.make_async_copy(
            vout.at[p], tblL.at[pl.ds(0, 128)], wsems[p]).wait()

      @pl.loop(0, _D, unroll=8)
      def _d(d):
        dcol = zeros + d
        for q in range(8):
          v = vin[p, d, pl.ds(q * _LANES, _LANES)]
          plsc.store_scatter(vout.at[p], [rows_q[q], dcol], v)

      pltpu.async_copy(vout.at[p], tblL.at[pl.ds(b * 128, 128)], wsems[p])

  issue(0, 0)

  @pl.loop(0, _SLOT_PAIRS)
  def _pair(k2):
    k0 = k2 * 2
    for half in range(2):
      k = k0 + half
      p = half
      issue(k + 1, 1 - p)
      process(k, p)

  pltpu.make_async_copy(vout.at[0], tblL.at[pl.ds(0, 128)], wsem0).wait()
  pltpu.make_async_copy(vout.at[1], tblL.at[pl.ds(0, 128)], wsem1).wait()


def _g_body(xT, tblL, posT, out, idx_v, posw, posv2, rows, slab,
            gsem0, gsem1, ssem0, ssem1):
  w = _wid()
  lane0 = w * _SEQ_W
  gsems = (gsem0, gsem1)
  ssems = (ssem0, ssem1)
  rows_q = _row_bases()
  zeros = jnp.zeros((_LANES,), jnp.int32)

  pltpu.sync_copy(xT.at[:, pl.ds(lane0, _SEQ_W)], idx_v)
  pltpu.sync_copy(posT, posw)

  @pl.loop(0, _D, unroll=8)
  def _pd(d):
    dcol = zeros + d
    for q in range(13):
      o = min(q * _LANES, _L - _LANES)
      v = posw[d, pl.ds(o, _LANES)]
      plsc.store_scatter(posv2, [rows_q[0] + o, dcol], v)

  def issue(l, p):
    @pl.when(l < _L)
    def _():
      pltpu.async_copy(tblL.at[idx_v.at[l]], rows.at[p], gsems[p])

  def process(l, p):
    pltpu.make_async_copy(
        tblL.at[pl.ds(0, _SEQ_W)], rows.at[p], gsems[p]).wait()

    @pl.when(l >= 2)
    def _():
      pltpu.make_async_copy(
          slab.at[p], out.at[0, :, pl.ds(0, _SEQ_W)], ssems[p]).wait()

    pvecs = [posv2[l, pl.ds(q * _LANES, _LANES)] for q in range(_D // _LANES)]

    @pl.loop(0, _SEQ_W, unroll=4)
    def _tok(j):
      jcol = zeros + j
      for q in range(_D // _LANES):
        v = rows[p, j, pl.ds(q * _LANES, _LANES)]
        plsc.store_scatter(slab.at[p], [rows_q[q], jcol], v + pvecs[q])

    pltpu.async_copy(slab.at[p], out.at[l, :, pl.ds(lane0, _SEQ_W)], ssems[p])

  issue(0, 0)

  @pl.loop(0, _L // 2)
  def _pair(l2):
    l0 = l2 * 2
    for half in range(2):
      l = l0 + half
      p = half
      issue(l + 1, 1 - p)
      process(l, p)

  pltpu.make_async_copy(
      slab.at[0], out.at[0, :, pl.ds(0, _SEQ_W)], ssem0).wait()
  pltpu.make_async_copy(
      slab.at[1], out.at[0, :, pl.ds(0, _SEQ_W)], ssem1).wait()


@jax.jit
def _run(x, token_table, pos_table):
  mesh = plsc.VectorSubcoreMesh(**_MESH)
  tblT = token_table.T
  xT = x.T
  posT = pos_table.T
  tailP = jnp.pad(token_table[_NBLK * 128:], ((0, 0), (0, 128 - _D)))

  t_kern = pl.kernel(
      _t_body,
      out_type=jax.ShapeDtypeStruct((_V, 128), jnp.float32),
      mesh=mesh,
      scratch_types=[
          pltpu.VMEM((2, _D, 128), jnp.float32),
          pltpu.VMEM((2, 128, 128), jnp.float32),
          pltpu.SemaphoreType.DMA,
          pltpu.SemaphoreType.DMA,
          pltpu.SemaphoreType.DMA,
          pltpu.SemaphoreType.DMA,
      ],
      compiler_params=_PARAMS,
  )
  tblL = t_kern(tblT, tailP)

  g_kern = pl.kernel(
      _g_body,
      out_type=jax.ShapeDtypeStruct((_L, _D, _B), jnp.float32),
      mesh=mesh,
      scratch_types=[
          pltpu.VMEM((_L, _SEQ_W), jnp.int32),
          pltpu.VMEM((_D, _L), jnp.float32),
          pltpu.VMEM((_L, _D), jnp.float32),
          pltpu.VMEM((2, _SEQ_W, 128), jnp.float32),
          pltpu.VMEM((2, _D, _SEQ_W), jnp.float32),
          pltpu.SemaphoreType.DMA,
          pltpu.SemaphoreType.DMA,
          pltpu.SemaphoreType.DMA,
          pltpu.SemaphoreType.DMA,
      ],
      compiler_params=_PARAMS,
  )
  outK = g_kern(xT, tblL, posT)
  return jnp.transpose(outK, (2, 0, 1))


def kernel(x, token_table, pos_table):
  return _run(x, token_table, pos_table[:_L])


# speedup vs baseline: 2.0169x; 1.4858x over previous
import jax
import jax.numpy as jnp
from jax import lax
from jax.experimental import pallas as pl
from jax.experimental.pallas import tpu as pltpu
from jax.experimental.pallas import tpu_sc as plsc

_V = 1000000
_D = 64
_B = 4096
_L = 200

_NC = 2
_NS = 16
_NW = _NC * _NS
_LANES = 16
_NBLK = _V // 128
_TAIL = _V - _NBLK * 128
_SLOT_PAIRS = (_NBLK // _NW + 2) // 2
_SEQ_W = _B // _NW

_MESH = dict(core_axis_name="c", subcore_axis_name="s",
             num_cores=_NC, num_subcores=_NS)
_PARAMS = pltpu.CompilerParams(
    use_tc_tiling_on_sc=True, needs_layout_passes=False)


def _wid():
  return lax.axis_index("s") * _NC + lax.axis_index("c")


def _row_bases():
  base = lax.iota(jnp.int32, _LANES)
  return [base + q * _LANES for q in range(8)]


def _t_body(tblT, tailP, tblL, vin, vout, gsem0, gsem1, wsem0, wsem1):
  w = _wid()
  gsems = (gsem0, gsem1)
  wsems = (wsem0, wsem1)
  rows_q = _row_bases()
  zeros = jnp.zeros((_LANES,), jnp.int32)

  @pl.when(w == 0)
  def _tail():
    pltpu.sync_copy(tailP, tblL.at[pl.ds(_NBLK * 128, _TAIL)])

  def issue(k, p):
    b = w + k * _NW

    @pl.when(b < _NBLK)
    def _():
      pltpu.async_copy(tblT.at[:, pl.ds(b * 128, 128)], vin.at[p], gsems[p])

  def process(k, p):
    b = w + k * _NW

    @pl.when(b < _NBLK)
    def _():
      pltpu.make_async_copy(
          tblT.at[:, pl.ds(0, 128)], vin.at[p], gsems[p]).wait()

      @pl.when(k >= 2)
      def _():
        pltpu.make_async_copy(
            vout.at[p], tblL.at[pl.ds(0, 128)], wsems[p]).wait()

      @plsc.parallel_loop(0, _D, unroll=8)
      def _d(d):
        dcol = zeros + d
        for q in range(8):
          v = vin[p, d, pl.ds(q * _LANES, _LANES)]
          plsc.store_scatter(vout.at[p], [rows_q[q], dcol], v)

      pltpu.async_copy(vout.at[p], tblL.at[pl.ds(b * 128, 128)], wsems[p])

  issue(0, 0)

  @pl.loop(0, _SLOT_PAIRS)
  def _pair(k2):
    k0 = k2 * 2
    for half in range(2):
      k = k0 + half
      p = half
      issue(k + 1, 1 - p)
      process(k, p)

  pltpu.make_async_copy(vout.at[0], tblL.at[pl.ds(0, 128)], wsem0).wait()
  pltpu.make_async_copy(vout.at[1], tblL.at[pl.ds(0, 128)], wsem1).wait()


def _g_body(xT, tblL, posT, out, idx_v, posw, posv2, rows, slab,
            gsem0, gsem1, ssem0, ssem1):
  w = _wid()
  lane0 = w * _SEQ_W
  gsems = (gsem0, gsem1)
  ssems = (ssem0, ssem1)
  rows_q = _row_bases()
  zeros = jnp.zeros((_LANES,), jnp.int32)

  pltpu.sync_copy(xT.at[:, pl.ds(lane0, _SEQ_W)], idx_v)
  pltpu.sync_copy(posT, posw)

  @plsc.parallel_loop(0, _D, unroll=8)
  def _pd(d):
    dcol = zeros + d
    for q in range(13):
      o = min(q * _LANES, _L - _LANES)
      v = posw[d, pl.ds(o, _LANES)]
      plsc.store_scatter(posv2, [rows_q[0] + o, dcol], v)

  def issue(l, p):
    @pl.when(l < _L)
    def _():
      pltpu.async_copy(tblL.at[idx_v.at[l]], rows.at[p], gsems[p])

  def process(l, p):
    pltpu.make_async_copy(
        tblL.at[pl.ds(0, _SEQ_W)], rows.at[p], gsems[p]).wait()

    @pl.when(l >= 2)
    def _():
      pltpu.make_async_copy(
          slab.at[p], out.at[0, :, pl.ds(0, _SEQ_W)], ssems[p]).wait()

    pvecs = [posv2[l, pl.ds(q * _LANES, _LANES)] for q in range(_D // _LANES)]

    @plsc.parallel_loop(0, _SEQ_W, unroll=8)
    def _tok(j):
      jcol = zeros + j
      for q in range(_D // _LANES):
        v = rows[p, j, pl.ds(q * _LANES, _LANES)]
        plsc.store_scatter(slab.at[p], [rows_q[q], jcol], v + pvecs[q])

    pltpu.async_copy(slab.at[p], out.at[l, :, pl.ds(lane0, _SEQ_W)], ssems[p])

  issue(0, 0)

  @pl.loop(0, _L // 2)
  def _pair(l2):
    l0 = l2 * 2
    for half in range(2):
      l = l0 + half
      p = half
      issue(l + 1, 1 - p)
      process(l, p)

  pltpu.make_async_copy(
      slab.at[0], out.at[0, :, pl.ds(0, _SEQ_W)], ssem0).wait()
  pltpu.make_async_copy(
      slab.at[1], out.at[0, :, pl.ds(0, _SEQ_W)], ssem1).wait()


@jax.jit
def _run(x, token_table, pos_table):
  mesh = plsc.VectorSubcoreMesh(**_MESH)
  tblT = token_table.T
  xT = x.T
  posT = pos_table.T
  tailP = jnp.pad(token_table[_NBLK * 128:], ((0, 0), (0, 128 - _D)))

  t_kern = pl.kernel(
      _t_body,
      out_type=jax.ShapeDtypeStruct((_V, 128), jnp.float32),
      mesh=mesh,
      scratch_types=[
          pltpu.VMEM((2, _D, 128), jnp.float32),
          pltpu.VMEM((2, 128, 128), jnp.float32),
          pltpu.SemaphoreType.DMA,
          pltpu.SemaphoreType.DMA,
          pltpu.SemaphoreType.DMA,
          pltpu.SemaphoreType.DMA,
      ],
      compiler_params=_PARAMS,
  )
  tblL = t_kern(tblT, tailP)

  g_kern = pl.kernel(
      _g_body,
      out_type=jax.ShapeDtypeStruct((_L, _D, _B), jnp.float32),
      mesh=mesh,
      scratch_types=[
          pltpu.VMEM((_L, _SEQ_W), jnp.int32),
          pltpu.VMEM((_D, _L), jnp.float32),
          pltpu.VMEM((_L, _D), jnp.float32),
          pltpu.VMEM((2, _SEQ_W, 128), jnp.float32),
          pltpu.VMEM((2, _D, _SEQ_W), jnp.float32),
          pltpu.SemaphoreType.DMA,
          pltpu.SemaphoreType.DMA,
          pltpu.SemaphoreType.DMA,
          pltpu.SemaphoreType.DMA,
      ],
      compiler_params=_PARAMS,
  )
  outK = g_kern(xT, tblL, posT)
  return jnp.transpose(outK, (2, 0, 1))


def kernel(x, token_table, pos_table):
  return _run(x, token_table, pos_table[:_L])
